# Initial kernel scaffold; baseline (speedup 1.0000x reference)
import jax
import jax.numpy as jnp
from jax.experimental import pallas as pl


def kernel(pointsSphere, colors, isoVert):
    raise NotImplementedError("write your pallas kernel here")


# speedup vs baseline: 10.7255x; 10.7255x over previous
import functools

import jax
import jax.numpy as jnp
from jax.experimental import pallas as pl
from jax.experimental.pallas import tpu as pltpu

_R_THRESH = 0.3
_BLOCK = 4096


def _blend_block(pts_ref, pcolt_ref, vt_ref, out_ref):
    pts = pts_ref[...]
    vt = vt_ref[...]
    t = jnp.dot(pts, vt, precision=jax.lax.Precision.HIGHEST,
                preferred_element_type=jnp.float32)
    d = jnp.sqrt(jnp.maximum(2.0 - 2.0 * t, 0.0))
    mask = d < _R_THRESH
    a = jnp.where(mask, d, 1.0)
    s = a
    num_p = s.shape[0]
    shift = 1
    while shift < num_p:
        ones = jnp.ones((shift, s.shape[1]), jnp.float32)
        s = s * jnp.concatenate([s[shift:, :], ones], axis=0)
        shift *= 2
    suffix = jnp.concatenate(
        [s[1:, :], jnp.ones((1, s.shape[1]), jnp.float32)], axis=0)
    w = jnp.where(mask, (1.0 - d) * suffix, 0.0)
    out_ref[...] = jnp.dot(pcolt_ref[...], w,
                           precision=jax.lax.Precision.HIGHEST,
                           preferred_element_type=jnp.float32)


@functools.partial(jax.jit, static_argnames=())
def kernel(pointsSphere, colors, isoVert):
    theta = pointsSphere[:, 0]
    phi = pointsSphere[:, 1]
    points = jnp.stack([jnp.sin(theta) * jnp.cos(phi),
                        jnp.sin(theta) * jnp.sin(phi),
                        jnp.cos(theta)], axis=1)
    num_p = points.shape[0]
    pcol = colors[jnp.arange(num_p) % colors.shape[0]]
    n = isoVert.shape[0]
    block = _BLOCK
    grid = (n + block - 1) // block
    vt = isoVert.T
    out = pl.pallas_call(
        _blend_block,
        grid=(grid,),
        in_specs=[
            pl.BlockSpec((num_p, 3), lambda i: (0, 0)),
            pl.BlockSpec((3, num_p), lambda i: (0, 0)),
            pl.BlockSpec((3, block), lambda i: (0, i)),
        ],
        out_specs=pl.BlockSpec((3, block), lambda i: (0, i)),
        out_shape=jax.ShapeDtypeStruct((3, n), jnp.float32),
        compiler_params=pltpu.CompilerParams(
            dimension_semantics=("parallel",)),
    )(points, pcol.T, vt)
    return out.T


# speedup vs baseline: 11.8608x; 1.1058x over previous
import functools

import jax
import jax.numpy as jnp
from jax.experimental import pallas as pl
from jax.experimental.pallas import tpu as pltpu

_R_THRESH = 0.3
_BLOCK = 8192


def _blend_block(pts_ref, pcolt_ref, vt_ref, out_ref):
    pts = pts_ref[...]
    vt = vt_ref[...]
    t = jnp.dot(pts, vt, precision=jax.lax.Precision.HIGHEST,
                preferred_element_type=jnp.float32)
    d = jnp.sqrt(jnp.maximum(2.0 - 2.0 * t, 0.0))
    mask = d < _R_THRESH
    a = jnp.where(mask, d, 1.0)
    s = a
    num_p = s.shape[0]
    shift = 1
    while shift < num_p:
        ones = jnp.ones((shift, s.shape[1]), jnp.float32)
        s = s * jnp.concatenate([s[shift:, :], ones], axis=0)
        shift *= 2
    suffix = jnp.concatenate(
        [s[1:, :], jnp.ones((1, s.shape[1]), jnp.float32)], axis=0)
    w = jnp.where(mask, (1.0 - d) * suffix, 0.0)
    out_ref[...] = jnp.dot(pcolt_ref[...], w,
                           precision=jax.lax.Precision.HIGHEST,
                           preferred_element_type=jnp.float32)


@functools.partial(jax.jit, static_argnames=())
def kernel(pointsSphere, colors, isoVert):
    theta = pointsSphere[:, 0]
    phi = pointsSphere[:, 1]
    points = jnp.stack([jnp.sin(theta) * jnp.cos(phi),
                        jnp.sin(theta) * jnp.sin(phi),
                        jnp.cos(theta)], axis=1)
    num_p = points.shape[0]
    pcol = colors[jnp.arange(num_p) % colors.shape[0]]
    n = isoVert.shape[0]
    block = _BLOCK
    grid = (n + block - 1) // block
    vt = isoVert.T
    out = pl.pallas_call(
        _blend_block,
        grid=(grid,),
        in_specs=[
            pl.BlockSpec((num_p, 3), lambda i: (0, 0)),
            pl.BlockSpec((3, num_p), lambda i: (0, 0)),
            pl.BlockSpec((3, block), lambda i: (0, i)),
        ],
        out_specs=pl.BlockSpec((3, block), lambda i: (0, i)),
        out_shape=jax.ShapeDtypeStruct((3, n), jnp.float32),
        compiler_params=pltpu.CompilerParams(
            dimension_semantics=("parallel",)),
    )(points, pcol.T, vt)
    return out.T


# speedup vs baseline: 13.0953x; 1.1041x over previous
import functools

import jax
import jax.numpy as jnp
from jax.experimental import pallas as pl
from jax.experimental.pallas import tpu as pltpu

_R_THRESH = 0.3
_BLOCK = 8192


def _blend_block(pts_ref, pcolt_ref, vt_ref, out_ref):
    pts = pts_ref[...]
    vt = vt_ref[...]
    t = jnp.dot(pts, vt, precision=jax.lax.Precision.HIGHEST,
                preferred_element_type=jnp.float32)
    z = jnp.maximum(2.0 - 2.0 * t, 1e-12)
    mask = z < _R_THRESH * _R_THRESH
    d = jnp.sqrt(z)
    a = jnp.where(mask, d, 1.0)
    s = a
    num_p = s.shape[0]
    shift = 1
    while shift < num_p:
        s = jnp.concatenate(
            [s[:num_p - shift, :] * s[shift:, :], s[num_p - shift:, :]],
            axis=0)
        shift *= 2
    suffix = jnp.concatenate(
        [s[1:, :], jnp.ones((1, s.shape[1]), jnp.float32)], axis=0)
    w = jnp.where(mask, (1.0 - d) * suffix, 0.0)
    nc = pcolt_ref.shape[1]
    g = w[0:nc] + w[nc:2 * nc] + w[2 * nc:3 * nc] + w[3 * nc:4 * nc]
    out_ref[...] = jnp.dot(pcolt_ref[...], g,
                           precision=jax.lax.Precision.HIGHEST,
                           preferred_element_type=jnp.float32)


@functools.partial(jax.jit, static_argnames=())
def kernel(pointsSphere, colors, isoVert):
    theta = pointsSphere[:, 0]
    phi = pointsSphere[:, 1]
    points = jnp.stack([jnp.sin(theta) * jnp.cos(phi),
                        jnp.sin(theta) * jnp.sin(phi),
                        jnp.cos(theta)], axis=1)
    num_p = points.shape[0]
    nc = colors.shape[0]
    n = isoVert.shape[0]
    block = _BLOCK
    grid = (n + block - 1) // block
    vt = isoVert.T
    out = pl.pallas_call(
        _blend_block,
        grid=(grid,),
        in_specs=[
            pl.BlockSpec((num_p, 3), lambda i: (0, 0)),
            pl.BlockSpec((3, nc), lambda i: (0, 0)),
            pl.BlockSpec((3, block), lambda i: (0, i)),
        ],
        out_specs=pl.BlockSpec((3, block), lambda i: (0, i)),
        out_shape=jax.ShapeDtypeStruct((3, n), jnp.float32),
        compiler_params=pltpu.CompilerParams(
            dimension_semantics=("parallel",)),
    )(points, colors.T, vt)
    return out.T
